# baseline (speedup 1.0000x reference)
import functools

import jax
import jax.numpy as jnp
from jax import lax
from jax.experimental import pallas as pl
from jax.experimental.pallas import tpu as pltpu
from jax.experimental.pallas import tpu_sc as plsc

_B, _S, _D, _E, _FF = 2, 2048, 768, 8, 3072
_T = _B * _S
_C = int((_T // _E) * 1.25)

_NW = 32
_TPW = _T // _NW
_CH = 64
_NCH = _TPW // _CH

_CP = _C + 8
_ECP = _E * _CP

_EP = 128
_TBLK = 512
_NT = _T // _TBLK
_FFT = 3072
_NF = _FF // _FFT


def _routing_body(x_ref, gw_ref, slot_ref, xs_ref, cnt_ref):
    i = pl.program_id(0)

    @pl.when(i == 0)
    def _():
        cnt_ref[...] = jnp.zeros_like(cnt_ref)

    x = x_ref[...]
    g = gw_ref[...]
    logits = jnp.dot(x, g, preferred_element_type=jnp.float32)
    col = lax.broadcasted_iota(jnp.int32, (_TBLK, _EP), 1)
    lm = jnp.where(col < _E, logits, jnp.float32(-1e30))
    m = jnp.max(lm, axis=1, keepdims=True)
    e_idx = jnp.min(jnp.where(lm == m, col, _EP), axis=1)
    denom = jnp.sum(jnp.where(col < _E, jnp.exp(lm - m), 0.0), axis=1)
    gval = 1.0 / denom
    mask_f = jnp.where(col == e_idx[:, None], 1.0, 0.0)
    r_i = lax.broadcasted_iota(jnp.int32, (_TBLK, _TBLK), 0)
    c_i = lax.broadcasted_iota(jnp.int32, (_TBLK, _TBLK), 1)
    tri = jnp.where(r_i > c_i, 1.0, 0.0)
    cum = jnp.dot(tri, mask_f, preferred_element_type=jnp.float32)
    pos = jnp.sum((cum + cnt_ref[...]) * mask_f, axis=1).astype(jnp.int32)
    cnt_ref[...] = cnt_ref[...] + jnp.sum(mask_f, axis=0, keepdims=True)
    keep = pos < _C
    tok = i * _TBLK + lax.broadcasted_iota(jnp.int32, (_TBLK,), 0)
    w = tok // _TPW
    dump = (w // 4) * _CP + _C + (w % 4)
    slot = jnp.where(keep, e_idx * _CP + pos, dump)
    slot_ref[0, 0, :] = slot
    xs_ref[...] = _pack_rows(x * jnp.where(keep, gval, 0.0)[:, None])


def _routing(tokens, gw_pad):
    return pl.pallas_call(
        _routing_body,
        grid=(_NT,),
        in_specs=[
            pl.BlockSpec((_TBLK, _D), lambda i: (i, 0)),
            pl.BlockSpec((_D, _EP), lambda i: (0, 0)),
        ],
        out_specs=[
            pl.BlockSpec((1, 1, _TBLK), lambda i: (i, 0, 0)),
            pl.BlockSpec((_TBLK, _D // 2), lambda i: (i, 0)),
        ],
        out_shape=[
            jax.ShapeDtypeStruct((_NT, 1, _TBLK), jnp.int32),
            jax.ShapeDtypeStruct((_T, _D // 2), jnp.int32),
        ],
        scratch_shapes=[pltpu.VMEM((1, _EP), jnp.float32)],
    )(tokens, gw_pad)


def _pack_rows(xf):
    bits = lax.bitcast_convert_type(xf, jnp.int32)
    rne = lambda b: b + 0x7FFF + ((b >> 16) & 1)
    half = xf.shape[-1] // 2
    lo16 = (rne(bits[:, :half]) >> 16) & 0xFFFF
    hi16 = rne(bits[:, half:]) & jnp.int32(-65536)
    return hi16 | lo16


def _unpack_rows(xi):
    return jnp.concatenate(
        [lax.bitcast_convert_type(xi << 16, jnp.float32),
         lax.bitcast_convert_type(xi & jnp.int32(-65536), jnp.float32)],
        axis=1)


def _ffn_body(x_ref, w1_ref, w2_ref, o_ref):
    x = _unpack_rows(x_ref[...])
    h = jnp.maximum(
        jnp.dot(x, w1_ref[0], preferred_element_type=jnp.float32), 0.0)
    o = jnp.dot(h, w2_ref[0], preferred_element_type=jnp.float32)
    o_ref[...] = o


def _ffn(disp, w1, w2):
    assert _NF == 1
    return pl.pallas_call(
        _ffn_body,
        grid=(_E,),
        in_specs=[
            pl.BlockSpec((_CP, _D // 2), lambda e: (e, 0)),
            pl.BlockSpec((1, _D, _FFT), lambda e: (e, 0, 0)),
            pl.BlockSpec((1, _FFT, _D), lambda e: (e, 0, 0)),
        ],
        out_specs=pl.BlockSpec((_CP, _D), lambda e: (e, 0)),
        out_shape=jax.ShapeDtypeStruct((_ECP, _D), jnp.float32),
    )(disp, w1, w2)


def _make_dispatch_sc():
    mesh = plsc.VectorSubcoreMesh(core_axis_name="c", subcore_axis_name="s")

    @functools.partial(
        pl.kernel,
        mesh=mesh,
        out_type=jax.ShapeDtypeStruct((_ECP, _D // 2), jnp.int32),
        scratch_types=[
            pltpu.VMEM((_NCH, _CH), jnp.int32),
            pltpu.VMEM((_CH, _D // 2), jnp.int32),
            pltpu.VMEM((_CH, _D // 2), jnp.int32),
            pltpu.SemaphoreType.DMA,
            pltpu.SemaphoreType.DMA,
        ],
    )
    def dispatch(slot_hbm, xs_hbm, disp_hbm, slot_v, rows_a, rows_b,
                 lsem, ssem):
        wid = lax.axis_index("s") * 2 + lax.axis_index("c")
        base = wid * _TPW
        pltpu.sync_copy(slot_hbm.at[wid], slot_v)
        bufs = (rows_a, rows_b)
        loads = [pltpu.async_copy(
            xs_hbm.at[pl.ds(base, _CH)], bufs[0], lsem)]
        stores = [None, None]
        for c in range(_NCH):
            b = c % 2
            if c + 1 < _NCH:
                nb = (c + 1) % 2
                if stores[nb] is not None:
                    stores[nb].wait()
                    stores[nb] = None
                loads.append(pltpu.async_copy(
                    xs_hbm.at[pl.ds(base + (c + 1) * _CH, _CH)],
                    bufs[nb], lsem))
            loads[c].wait()
            stores[b] = pltpu.async_copy(
                bufs[b], disp_hbm.at[slot_v.at[c]], ssem)
        for st in stores:
            if st is not None:
                st.wait()

    return dispatch


def _make_combine_sc():
    mesh = plsc.VectorSubcoreMesh(core_axis_name="c", subcore_axis_name="s")

    @functools.partial(
        pl.kernel,
        mesh=mesh,
        out_type=jax.ShapeDtypeStruct((_T, _D), jnp.float32),
        scratch_types=[
            pltpu.VMEM((_NCH, _CH), jnp.int32),
            pltpu.VMEM((_CH, _D), jnp.float32),
            pltpu.VMEM((_CH, _D), jnp.float32),
            pltpu.SemaphoreType.DMA,
            pltpu.SemaphoreType.DMA,
        ],
    )
    def combine(slot_hbm, eo_hbm, out_hbm, slot_v, rows_a, rows_b,
                gsem, ssem):
        wid = lax.axis_index("s") * 2 + lax.axis_index("c")
        base = wid * _TPW
        pltpu.sync_copy(slot_hbm.at[wid], slot_v)
        bufs = (rows_a, rows_b)
        gaths = [pltpu.async_copy(
            eo_hbm.at[slot_v.at[0]], bufs[0], gsem)]
        stores = [None, None]
        for c in range(_NCH):
            b = c % 2
            if c + 1 < _NCH:
                nb = (c + 1) % 2
                if stores[nb] is not None:
                    stores[nb].wait()
                    stores[nb] = None
                gaths.append(pltpu.async_copy(
                    eo_hbm.at[slot_v.at[c + 1]], bufs[nb], gsem))
            gaths[c].wait()
            stores[b] = pltpu.async_copy(
                bufs[b], out_hbm.at[pl.ds(base + c * _CH, _CH)], ssem)
        for st in stores:
            if st is not None:
                st.wait()

    return combine


def kernel(inputs, gate_w, w1, w2):
    tokens = inputs.reshape(_T, _D)
    gw_pad = jnp.pad(gate_w, ((0, 0), (0, _EP - _E)))
    slot3, xs = _routing(tokens, gw_pad)
    slot_sc = slot3.reshape(_NW, _NCH, _CH)
    disp = _make_dispatch_sc()(slot_sc, xs)
    eo = _ffn(disp, w1, w2)
    out = _make_combine_sc()(slot_sc, eo)
    return out.reshape(inputs.shape)
